# baseline (speedup 1.0000x reference)
import functools

import jax
import jax.numpy as jnp
from jax import lax
from jax.experimental import pallas as pl
from jax.experimental.pallas import tpu as pltpu
from jax.experimental.pallas import tpu_sc as plsc

N = 10000
D = 128
E = 320000
NW = 32
EPW = E // NW
EPW_PAD = 10240
PAD_PER_W = EPW_PAD - EPW
EEB = 4

CHUNK = 32
NCHUNK = EPW_PAD // CHUNK
BLK = 16
NG = CHUNK // 16
NBUF = 3
NFS = 2

CHUNK2 = 128
NCHUNK2 = EPW_PAD // CHUNK2
BLK2 = 16
NG2 = CHUNK2 // 16


def _tc1_body(x_ref, w1_ref, a1s_ref, a1d_ref, h_ref, hb_ref, avt_ref):
    h = jnp.dot(x_ref[...], w1_ref[...], preferred_element_type=jnp.float32)
    h_ref[...] = h
    hb_ref[...] = h.astype(jnp.bfloat16)
    avt_ref[0, :] = jnp.dot(h, a1s_ref[...], preferred_element_type=jnp.float32)
    avt_ref[1, :] = jnp.dot(h, a1d_ref[...], preferred_element_type=jnp.float32)


def _tc2_body(p_ref, den_ref, w2_ref, a2s_ref, a2d_ref, tab_ref):
    i = pl.program_id(0)
    den = den_ref[i, :] + den_ref[10 + i, :] + 1e-16
    h1 = (p_ref[0] + p_ref[1]) / den[:, None]
    h1e = jnp.where(h1 > 0.0, h1, jnp.exp(h1) - 1.0)
    h2 = jnp.dot(h1e, w2_ref[...], preferred_element_type=jnp.float32)[:, 0]
    tab_ref[i, :] = h2
    tab_ref[10 + i, :] = h2 * a2s_ref[0]
    tab_ref[20 + i, :] = h2 * a2d_ref[0]


_MESH = plsc.VectorSubcoreMesh(core_axis_name="c", subcore_axis_name="s")


@functools.partial(
    pl.kernel,
    out_type=[
        jax.ShapeDtypeStruct((2 * N, D), jnp.float32),
        jax.ShapeDtypeStruct((2 * N,), jnp.float32),
    ],
    mesh=_MESH,
    compiler_params=pltpu.CompilerParams(needs_layout_passes=False,
                                         use_tc_tiling_on_sc=False),
    scratch_types=[
        pltpu.VMEM((N,), jnp.float32),
        pltpu.VMEM((N,), jnp.float32),
        pltpu.VMEM((2 * BLK, CHUNK), jnp.int32),
        pltpu.VMEM((2 * BLK, CHUNK), jnp.int32),
        pltpu.VMEM((EEB, CHUNK), jnp.float32),
        pltpu.VMEM((NBUF, CHUNK, D // 2), jnp.int32),
        pltpu.VMEM((NFS, CHUNK, D), jnp.float32),
        pltpu.VMEM((640,), jnp.float32),
        pltpu.VMEM_SHARED((N, D), jnp.float32),
        pltpu.VMEM_SHARED((N,), jnp.float32),
        pltpu.SemaphoreType.DMA,
        pltpu.SemaphoreType.DMA,
        pltpu.SemaphoreType.DMA,
    ],
)
def _sc1(hb_hbm, avt_hbm, srcp_hbm, dstp_hbm, out_hbm, den_hbm,
         as_t, ad_t, src_v, dst_v, ee_v, rows_b, rows_f, zed_v, acc_sh,
         den_sh, sem_g, sem_s, sem_d):
    cid = lax.axis_index("c")
    sid = lax.axis_index("s")
    w = sid * 2 + cid
    z16 = jnp.zeros((16,), jnp.float32)

    for k in range(40):
        zed_v[pl.ds(k * 16, 16)] = z16

    def _zrow(r, carry):
        for g in range(8):
            rows_f[0, r, pl.ds(g * 16, 16)] = z16
        return carry

    lax.fori_loop(0, CHUNK, _zrow, 0)

    d0 = sid * 624

    def _zacc(j, carry):
        o = pl.multiple_of(d0 + j * CHUNK, 16)
        pltpu.sync_copy(rows_f.at[0], acc_sh.at[pl.ds(o, CHUNK)])
        return carry

    lax.fori_loop(0, 20, _zacc, 0)

    @pl.when(sid < 15)
    def _():
        pltpu.sync_copy(zed_v.at[pl.ds(0, 624)], den_sh.at[pl.ds(d0, 624)])

    @pl.when(sid == 15)
    def _():
        pltpu.sync_copy(zed_v, den_sh.at[pl.ds(15 * 624, 640)])

    pltpu.sync_copy(avt_hbm.at[pl.ds(0, N)], as_t)
    pltpu.sync_copy(avt_hbm.at[pl.ds(N, N)], ad_t)
    w0 = pl.multiple_of(w * NCHUNK, BLK)
    pltpu.sync_copy(srcp_hbm.at[pl.ds(w0, BLK)], src_v.at[pl.ds(0, BLK)])
    pltpu.sync_copy(dstp_hbm.at[pl.ds(w0, BLK)], dst_v.at[pl.ds(0, BLK)])
    plsc.subcore_barrier()

    pltpu.async_copy(hb_hbm.at[src_v.at[0]], rows_b.at[0], sem_g)
    pltpu.async_copy(hb_hbm.at[src_v.at[1]], rows_b.at[1], sem_g)

    def _chunk(c, carry):
        bslot = lax.rem(c, NBUF)
        fslot = lax.rem(c, NFS)
        blkrow = lax.rem(lax.div(c, BLK), 2) * BLK + lax.rem(c, BLK)
        erow = lax.rem(c, EEB)

        @pl.when(c >= EEB)
        def _():
            pltpu.make_async_copy(
                ee_v.at[erow], den_sh.at[dst_v.at[blkrow]], sem_d).wait()

        for g in range(NG):
            sv = src_v[blkrow, pl.ds(g * 16, 16)]
            dv = dst_v[blkrow, pl.ds(g * 16, 16)]
            a_s = plsc.load_gather(as_t, [sv])
            a_d = plsc.load_gather(ad_t, [dv])
            e = a_s + a_d
            e = jnp.where(e < 0.0, e * 0.2, e)
            ee = jnp.exp(e)
            valid = (c * CHUNK + g * 16 + lax.iota(jnp.int32, 16)) < EPW
            ee_v[erow, pl.ds(g * 16, 16)] = jnp.where(valid, ee, 0.0)
        pltpu.async_copy(ee_v.at[erow], den_sh.at[dst_v.at[blkrow]],
                         sem_d, add=True)

        pltpu.make_async_copy(hb_hbm.at[src_v.at[blkrow]], rows_b.at[bslot],
                              sem_g).wait()

        @pl.when(c >= 2)
        def _():
            pltpu.make_async_copy(
                rows_f.at[fslot], acc_sh.at[dst_v.at[blkrow]], sem_s).wait()

        erow16 = jnp.broadcast_to(erow, (16,))

        @plsc.parallel_loop(0, CHUNK, unroll=8)
        def _scale(r):
            s = plsc.load_gather(ee_v, [erow16, jnp.broadcast_to(r, (16,))])
            for g in range(4):
                w = rows_b[bslot, r, pl.ds(g * 16, 16)]
                a = plsc.bitcast(jnp.left_shift(w, 16), jnp.float32)
                b = plsc.bitcast(
                    jnp.bitwise_and(w, jnp.int32(-65536)), jnp.float32)
                rows_f[fslot, r, pl.ds(g * 32, 16)] = a * s
                rows_f[fslot, r, pl.ds(g * 32 + 16, 16)] = b * s

        pltpu.async_copy(rows_f.at[fslot], acc_sh.at[dst_v.at[blkrow]],
                         sem_s, add=True)

        @pl.when(jnp.logical_and(lax.rem(c + 2, BLK) == 0, c + 2 < NCHUNK))
        def _():
            q = c + 2
            half = pl.multiple_of(lax.rem(lax.div(q, BLK), 2) * BLK, BLK)
            r0 = pl.multiple_of(w * NCHUNK + q, BLK)
            pltpu.sync_copy(srcp_hbm.at[pl.ds(r0, BLK)],
                            src_v.at[pl.ds(half, BLK)])
            pltpu.sync_copy(dstp_hbm.at[pl.ds(r0, BLK)],
                            dst_v.at[pl.ds(half, BLK)])

        @pl.when(c + 2 < NCHUNK)
        def _():
            q = c + 2
            nrow = lax.rem(lax.div(q, BLK), 2) * BLK + lax.rem(q, BLK)
            pltpu.async_copy(hb_hbm.at[src_v.at[nrow]],
                             rows_b.at[lax.rem(q, NBUF)], sem_g)

        return carry

    lax.fori_loop(0, NCHUNK, _chunk, 0)

    for k in range(2):
        pltpu.make_async_copy(rows_f.at[k], acc_sh.at[dst_v.at[0]],
                              sem_s).wait()
    for k in range(EEB):
        pltpu.make_async_copy(ee_v.at[k], den_sh.at[dst_v.at[0]],
                              sem_d).wait()
    plsc.subcore_barrier()

    @pl.when(sid < 15)
    def _():
        pltpu.sync_copy(acc_sh.at[pl.ds(d0, 624)],
                        out_hbm.at[pl.ds(cid * N + d0, 624)])
        pltpu.sync_copy(den_sh.at[pl.ds(d0, 624)], zed_v.at[pl.ds(0, 624)])
        pltpu.sync_copy(zed_v.at[pl.ds(0, 624)],
                        den_hbm.at[pl.ds(cid * N + d0, 624)])

    @pl.when(sid == 15)
    def _():
        pltpu.sync_copy(acc_sh.at[pl.ds(15 * 624, 640)],
                        out_hbm.at[pl.ds(cid * N + 15 * 624, 640)])
        pltpu.sync_copy(den_sh.at[pl.ds(15 * 624, 640)], zed_v)
        pltpu.sync_copy(zed_v,
                        den_hbm.at[pl.ds(cid * N + 15 * 624, 640)])


@functools.partial(
    pl.kernel,
    out_type=jax.ShapeDtypeStruct((N,), jnp.float32),
    mesh=_MESH,
    compiler_params=pltpu.CompilerParams(needs_layout_passes=False),
    scratch_types=[
        pltpu.VMEM((N,), jnp.float32),
        pltpu.VMEM((N,), jnp.float32),
        pltpu.VMEM((N,), jnp.float32),
        pltpu.VMEM((2 * BLK2, CHUNK2), jnp.int32),
        pltpu.VMEM((2 * BLK2, CHUNK2), jnp.int32),
        pltpu.VMEM((EEB, CHUNK2), jnp.float32),
        pltpu.VMEM((EEB, CHUNK2), jnp.float32),
        pltpu.VMEM((640,), jnp.float32),
        pltpu.VMEM((640,), jnp.float32),
        pltpu.VMEM((640,), jnp.float32),
        pltpu.VMEM_SHARED((N,), jnp.float32),
        pltpu.VMEM_SHARED((N,), jnp.float32),
        pltpu.SemaphoreType.DMA,
        pltpu.SemaphoreType.DMA,
    ],
)
def _sc2(tab_hbm, srcp_hbm, dstp_hbm, out_hbm,
         h2_t, as2_t, ad2_t, src_v, dst_v, ee_v, val_v,
         num_b, den_b, out_b, num_sh, den_sh, sem_n, sem_d):
    cid = lax.axis_index("c")
    sid = lax.axis_index("s")
    NC2 = 2 * NCHUNK2

    @pl.when(cid == 0)
    def _core0():
        z16 = jnp.zeros((16,), jnp.float32)
        for k in range(40):
            out_b[pl.ds(k * 16, 16)] = z16
        d0 = sid * 624

        @pl.when(sid < 15)
        def _():
            pltpu.sync_copy(out_b.at[pl.ds(0, 624)], num_sh.at[pl.ds(d0, 624)])
            pltpu.sync_copy(out_b.at[pl.ds(0, 624)], den_sh.at[pl.ds(d0, 624)])

        @pl.when(sid == 15)
        def _():
            pltpu.sync_copy(out_b, num_sh.at[pl.ds(15 * 624, 640)])
            pltpu.sync_copy(out_b, den_sh.at[pl.ds(15 * 624, 640)])

        pltpu.sync_copy(tab_hbm.at[pl.ds(0, N)], h2_t)
        pltpu.sync_copy(tab_hbm.at[pl.ds(N, N)], as2_t)
        pltpu.sync_copy(tab_hbm.at[pl.ds(2 * N, N)], ad2_t)
        s0 = pl.multiple_of(sid * NC2, BLK2)
        pltpu.sync_copy(srcp_hbm.at[pl.ds(s0, BLK2)], src_v.at[pl.ds(0, BLK2)])
        pltpu.sync_copy(dstp_hbm.at[pl.ds(s0, BLK2)], dst_v.at[pl.ds(0, BLK2)])
        plsc.subcore_barrier()

        def _chunk(c, carry):
            blkrow = lax.rem(lax.div(c, BLK2), 2) * BLK2 + lax.rem(c, BLK2)
            erow = lax.rem(c, EEB)
            cw = lax.rem(c, NCHUNK2)

            @pl.when(c >= EEB)
            def _():
                pltpu.make_async_copy(
                    val_v.at[erow], num_sh.at[dst_v.at[blkrow]], sem_n).wait()
                pltpu.make_async_copy(
                    ee_v.at[erow], den_sh.at[dst_v.at[blkrow]], sem_d).wait()

            @plsc.parallel_loop(0, NG2, unroll=4)
            def _grp(g):
                o = pl.multiple_of(g * 16, 16)
                sv = src_v[blkrow, pl.ds(o, 16)]
                dv = dst_v[blkrow, pl.ds(o, 16)]
                hs = plsc.load_gather(h2_t, [sv])
                a_s = plsc.load_gather(as2_t, [sv])
                a_d = plsc.load_gather(ad2_t, [dv])
                e = a_s + a_d
                e = jnp.where(e < 0.0, e * 0.2, e)
                ee = jnp.exp(e)
                valid = (cw * CHUNK2 + o + lax.iota(jnp.int32, 16)) < EPW
                ee = jnp.where(valid, ee, 0.0)
                ee_v[erow, pl.ds(o, 16)] = ee
                val_v[erow, pl.ds(o, 16)] = ee * hs
            pltpu.async_copy(val_v.at[erow], num_sh.at[dst_v.at[blkrow]],
                             sem_n, add=True)
            pltpu.async_copy(ee_v.at[erow], den_sh.at[dst_v.at[blkrow]],
                             sem_d, add=True)

            @pl.when(jnp.logical_and(lax.rem(c + 1, BLK2) == 0, c + 1 < NC2))
            def _():
                q = c + 1
                half = pl.multiple_of(
                    lax.rem(lax.div(q, BLK2), 2) * BLK2, BLK2)
                r0 = pl.multiple_of(sid * NC2 + q, BLK2)
                pltpu.sync_copy(srcp_hbm.at[pl.ds(r0, BLK2)],
                                src_v.at[pl.ds(half, BLK2)])
                pltpu.sync_copy(dstp_hbm.at[pl.ds(r0, BLK2)],
                                dst_v.at[pl.ds(half, BLK2)])

            return carry

        lax.fori_loop(0, NC2, _chunk, 0)
        for k in range(EEB):
            pltpu.make_async_copy(val_v.at[k], num_sh.at[dst_v.at[0]],
                                  sem_n).wait()
            pltpu.make_async_copy(ee_v.at[k], den_sh.at[dst_v.at[0]],
                                  sem_d).wait()
        plsc.subcore_barrier()

        @pl.when(sid < 15)
        def _():
            pltpu.sync_copy(num_sh.at[pl.ds(d0, 624)], num_b.at[pl.ds(0, 624)])
            pltpu.sync_copy(den_sh.at[pl.ds(d0, 624)], den_b.at[pl.ds(0, 624)])

        @pl.when(sid == 15)
        def _():
            pltpu.sync_copy(num_sh.at[pl.ds(15 * 624, 640)], num_b)
            pltpu.sync_copy(den_sh.at[pl.ds(15 * 624, 640)], den_b)

        for k in range(40):
            sl = pl.ds(k * 16, 16)
            out_b[sl] = num_b[sl] / (den_b[sl] + 1e-16)

        @pl.when(sid < 15)
        def _():
            pltpu.sync_copy(out_b.at[pl.ds(0, 624)], out_hbm.at[pl.ds(d0, 624)])

        @pl.when(sid == 15)
        def _():
            pltpu.sync_copy(out_b, out_hbm.at[pl.ds(15 * 624, 640)])


def kernel(x, edge_index, W1, a1_src, a1_dst, W2, a2_src, a2_dst):
    src = edge_index[0].astype(jnp.int32)
    dst = edge_index[1].astype(jnp.int32)
    pad = jnp.broadcast_to(
        (jnp.arange(PAD_PER_W, dtype=jnp.int32) * 37) % N, (NW, PAD_PER_W))
    srcp = jnp.concatenate([src.reshape(NW, EPW), pad], axis=1)
    srcp = srcp.reshape(NW * NCHUNK, CHUNK)
    dstp = jnp.concatenate([dst.reshape(NW, EPW), pad], axis=1)
    dstp = dstp.reshape(NW * NCHUNK, CHUNK)

    h, hb, avt = pl.pallas_call(
        _tc1_body,
        out_shape=[
            jax.ShapeDtypeStruct((N, D), jnp.float32),
            jax.ShapeDtypeStruct((N, D), jnp.bfloat16),
            jax.ShapeDtypeStruct((2, N), jnp.float32),
        ],
    )(x, W1, a1_src, a1_dst)

    hb32 = jax.lax.bitcast_convert_type(
        hb.reshape(N, D // 2, 2), jnp.int32)
    part, dens = _sc1(hb32, avt.reshape(2 * N), srcp, dstp)

    import numpy as _np
    _perm = _np.arange(D).reshape(D // 32, 16, 2).transpose(0, 2, 1).reshape(D)
    W2 = W2[jnp.asarray(_perm)]

    RB = 1000
    tab = pl.pallas_call(
        _tc2_body,
        grid=(N // RB,),
        in_specs=[
            pl.BlockSpec((2, RB, D), lambda i: (0, i, 0)),
            pl.BlockSpec((20, RB), lambda i: (0, 0)),
            pl.BlockSpec((D, 1), lambda i: (0, 0)),
            pl.BlockSpec(memory_space=pltpu.SMEM),
            pl.BlockSpec(memory_space=pltpu.SMEM),
        ],
        out_specs=pl.BlockSpec((30, RB), lambda i: (0, 0)),
        out_shape=jax.ShapeDtypeStruct((30, RB), jnp.float32),
    )(part.reshape(2, N, D), dens.reshape(20, RB), W2, a2_src, a2_dst)

    out2 = _sc2(tab.reshape(3 * N),
                srcp.reshape(NW * NCHUNK2, CHUNK2),
                dstp.reshape(NW * NCHUNK2, CHUNK2))
    return out2.reshape(1, N)


# speedup vs baseline: 1.0231x; 1.0231x over previous
import functools

import jax
import jax.numpy as jnp
from jax import lax
from jax.experimental import pallas as pl
from jax.experimental.pallas import tpu as pltpu
from jax.experimental.pallas import tpu_sc as plsc

N = 10000
D = 128
E = 320000
NW = 32
EPW = E // NW
EPW_PAD = 10240
PAD_PER_W = EPW_PAD - EPW
EEB = 4

CHUNK = 32
NCHUNK = EPW_PAD // CHUNK
BLK = 32
NG = CHUNK // 16
NBUF = 3

CHUNK2 = 128
NCHUNK2 = EPW_PAD // CHUNK2
BLK2 = 16
NG2 = CHUNK2 // 16


def _tc1_body(x_ref, w1_ref, a1s_ref, a1d_ref, h_ref, avt_ref):
    h = jnp.dot(x_ref[...], w1_ref[...], preferred_element_type=jnp.float32)
    h_ref[...] = h
    avt_ref[0, :] = jnp.dot(h, a1s_ref[...], preferred_element_type=jnp.float32)
    avt_ref[1, :] = jnp.dot(h, a1d_ref[...], preferred_element_type=jnp.float32)


def _tc2_body(p_ref, den_ref, w2_ref, a2s_ref, a2d_ref, tab_ref):
    i = pl.program_id(0)
    den = den_ref[i, :] + den_ref[10 + i, :] + 1e-16
    h1 = (p_ref[0] + p_ref[1]) / den[:, None]
    h1e = jnp.where(h1 > 0.0, h1, jnp.exp(h1) - 1.0)
    h2 = jnp.dot(h1e, w2_ref[...], preferred_element_type=jnp.float32)[:, 0]
    tab_ref[i, :] = h2
    tab_ref[10 + i, :] = h2 * a2s_ref[0]
    tab_ref[20 + i, :] = h2 * a2d_ref[0]


_MESH = plsc.VectorSubcoreMesh(core_axis_name="c", subcore_axis_name="s")


@functools.partial(
    pl.kernel,
    out_type=[
        jax.ShapeDtypeStruct((2 * N, D), jnp.float32),
        jax.ShapeDtypeStruct((2 * N,), jnp.float32),
    ],
    mesh=_MESH,
    compiler_params=pltpu.CompilerParams(needs_layout_passes=False),
    scratch_types=[
        pltpu.VMEM((N,), jnp.float32),
        pltpu.VMEM((N,), jnp.float32),
        pltpu.VMEM((2 * BLK, CHUNK), jnp.int32),
        pltpu.VMEM((2 * BLK, CHUNK), jnp.int32),
        pltpu.VMEM((EEB, CHUNK), jnp.float32),
        pltpu.VMEM((NBUF, CHUNK, D), jnp.float32),
        pltpu.VMEM((640,), jnp.float32),
        pltpu.VMEM_SHARED((N, D), jnp.float32),
        pltpu.VMEM_SHARED((N,), jnp.float32),
        pltpu.SemaphoreType.DMA,
        pltpu.SemaphoreType.DMA,
        pltpu.SemaphoreType.DMA,
    ],
)
def _sc1(h_hbm, avt_hbm, srcp_hbm, dstp_hbm, out_hbm, den_hbm,
         as_t, ad_t, src_v, dst_v, ee_v, rows_v, zed_v, acc_sh, den_sh,
         sem_g, sem_s, sem_d):
    cid = lax.axis_index("c")
    sid = lax.axis_index("s")
    w = sid * 2 + cid
    z16 = jnp.zeros((16,), jnp.float32)

    for k in range(40):
        zed_v[pl.ds(k * 16, 16)] = z16

    def _zrow(r, carry):
        for g in range(8):
            rows_v[0, r, pl.ds(g * 16, 16)] = z16
        return carry

    lax.fori_loop(0, CHUNK, _zrow, 0)

    d0 = sid * 624

    def _zacc(j, carry):
        o = pl.multiple_of(d0 + j * CHUNK, 16)
        pltpu.sync_copy(rows_v.at[0], acc_sh.at[pl.ds(o, CHUNK)])
        return carry

    lax.fori_loop(0, 20, _zacc, 0)

    @pl.when(sid < 15)
    def _():
        pltpu.sync_copy(zed_v.at[pl.ds(0, 624)], den_sh.at[pl.ds(d0, 624)])

    @pl.when(sid == 15)
    def _():
        pltpu.sync_copy(zed_v, den_sh.at[pl.ds(15 * 624, 640)])

    pltpu.sync_copy(avt_hbm.at[pl.ds(0, N)], as_t)
    pltpu.sync_copy(avt_hbm.at[pl.ds(N, N)], ad_t)
    w0 = pl.multiple_of(w * NCHUNK, BLK)
    pltpu.sync_copy(srcp_hbm.at[pl.ds(w0, BLK)], src_v.at[pl.ds(0, BLK)])
    pltpu.sync_copy(dstp_hbm.at[pl.ds(w0, BLK)], dst_v.at[pl.ds(0, BLK)])
    plsc.subcore_barrier()

    pltpu.async_copy(h_hbm.at[src_v.at[0]], rows_v.at[0], sem_g)
    pltpu.async_copy(h_hbm.at[src_v.at[1]], rows_v.at[1], sem_g)

    def _chunk(c, carry):
        buf = lax.rem(c, NBUF)
        blkrow = lax.rem(lax.div(c, BLK), 2) * BLK + lax.rem(c, BLK)
        erow = lax.rem(c, EEB)

        @pl.when(c >= EEB)
        def _():
            pltpu.make_async_copy(
                ee_v.at[erow], den_sh.at[dst_v.at[blkrow]], sem_d).wait()

        for g in range(NG):
            sv = src_v[blkrow, pl.ds(g * 16, 16)]
            dv = dst_v[blkrow, pl.ds(g * 16, 16)]
            a_s = plsc.load_gather(as_t, [sv])
            a_d = plsc.load_gather(ad_t, [dv])
            e = a_s + a_d
            e = jnp.where(e < 0.0, e * 0.2, e)
            ee = jnp.exp(e)
            valid = (c * CHUNK + g * 16 + lax.iota(jnp.int32, 16)) < EPW
            ee_v[erow, pl.ds(g * 16, 16)] = jnp.where(valid, ee, 0.0)
        pltpu.async_copy(ee_v.at[erow], den_sh.at[dst_v.at[blkrow]],
                         sem_d, add=True)

        pltpu.make_async_copy(h_hbm.at[src_v.at[blkrow]], rows_v.at[buf],
                              sem_g).wait()
        erow16 = jnp.broadcast_to(erow, (16,))

        @plsc.parallel_loop(0, CHUNK, unroll=8)
        def _scale(r):
            s = plsc.load_gather(ee_v, [erow16, jnp.broadcast_to(r, (16,))])
            for g in range(8):
                rows_v[buf, r, pl.ds(g * 16, 16)] = (
                    rows_v[buf, r, pl.ds(g * 16, 16)] * s)

        pltpu.async_copy(rows_v.at[buf], acc_sh.at[dst_v.at[blkrow]],
                         sem_s, add=True)

        @pl.when(c >= 1)
        def _():
            pltpu.make_async_copy(
                rows_v.at[lax.rem(c + 2, NBUF)],
                acc_sh.at[dst_v.at[blkrow]], sem_s).wait()

        @pl.when(jnp.logical_and(lax.rem(c + 2, BLK) == 0, c + 2 < NCHUNK))
        def _():
            q = c + 2
            half = pl.multiple_of(lax.rem(lax.div(q, BLK), 2) * BLK, BLK)
            r0 = pl.multiple_of(w * NCHUNK + q, BLK)
            pltpu.sync_copy(srcp_hbm.at[pl.ds(r0, BLK)],
                            src_v.at[pl.ds(half, BLK)])
            pltpu.sync_copy(dstp_hbm.at[pl.ds(r0, BLK)],
                            dst_v.at[pl.ds(half, BLK)])

        @pl.when(c + 2 < NCHUNK)
        def _():
            q = c + 2
            nrow = lax.rem(lax.div(q, BLK), 2) * BLK + lax.rem(q, BLK)
            pltpu.async_copy(h_hbm.at[src_v.at[nrow]],
                             rows_v.at[lax.rem(q, NBUF)], sem_g)

        return carry

    lax.fori_loop(0, NCHUNK, _chunk, 0)

    pltpu.make_async_copy(rows_v.at[0], acc_sh.at[dst_v.at[0]],
                          sem_s).wait()
    for k in range(EEB):
        pltpu.make_async_copy(ee_v.at[k], den_sh.at[dst_v.at[0]],
                              sem_d).wait()
    plsc.subcore_barrier()

    @pl.when(sid < 15)
    def _():
        pltpu.sync_copy(acc_sh.at[pl.ds(d0, 624)],
                        out_hbm.at[pl.ds(cid * N + d0, 624)])
        pltpu.sync_copy(den_sh.at[pl.ds(d0, 624)], zed_v.at[pl.ds(0, 624)])
        pltpu.sync_copy(zed_v.at[pl.ds(0, 624)],
                        den_hbm.at[pl.ds(cid * N + d0, 624)])

    @pl.when(sid == 15)
    def _():
        pltpu.sync_copy(acc_sh.at[pl.ds(15 * 624, 640)],
                        out_hbm.at[pl.ds(cid * N + 15 * 624, 640)])
        pltpu.sync_copy(den_sh.at[pl.ds(15 * 624, 640)], zed_v)
        pltpu.sync_copy(zed_v,
                        den_hbm.at[pl.ds(cid * N + 15 * 624, 640)])


@functools.partial(
    pl.kernel,
    out_type=jax.ShapeDtypeStruct((N,), jnp.float32),
    mesh=_MESH,
    compiler_params=pltpu.CompilerParams(needs_layout_passes=False),
    scratch_types=[
        pltpu.VMEM((N,), jnp.float32),
        pltpu.VMEM((N,), jnp.float32),
        pltpu.VMEM((N,), jnp.float32),
        pltpu.VMEM((2 * BLK2, CHUNK2), jnp.int32),
        pltpu.VMEM((2 * BLK2, CHUNK2), jnp.int32),
        pltpu.VMEM((EEB, CHUNK2), jnp.float32),
        pltpu.VMEM((EEB, CHUNK2), jnp.float32),
        pltpu.VMEM((640,), jnp.float32),
        pltpu.VMEM((640,), jnp.float32),
        pltpu.VMEM((640,), jnp.float32),
        pltpu.VMEM_SHARED((N,), jnp.float32),
        pltpu.VMEM_SHARED((N,), jnp.float32),
        pltpu.SemaphoreType.DMA,
        pltpu.SemaphoreType.DMA,
    ],
)
def _sc2(tab_hbm, srcp_hbm, dstp_hbm, out_hbm,
         h2_t, as2_t, ad2_t, src_v, dst_v, ee_v, val_v,
         num_b, den_b, out_b, num_sh, den_sh, sem_n, sem_d):
    cid = lax.axis_index("c")
    sid = lax.axis_index("s")
    NC2 = 2 * NCHUNK2

    @pl.when(cid == 0)
    def _core0():
        z16 = jnp.zeros((16,), jnp.float32)
        for k in range(40):
            out_b[pl.ds(k * 16, 16)] = z16
        d0 = sid * 624

        @pl.when(sid < 15)
        def _():
            pltpu.sync_copy(out_b.at[pl.ds(0, 624)], num_sh.at[pl.ds(d0, 624)])
            pltpu.sync_copy(out_b.at[pl.ds(0, 624)], den_sh.at[pl.ds(d0, 624)])

        @pl.when(sid == 15)
        def _():
            pltpu.sync_copy(out_b, num_sh.at[pl.ds(15 * 624, 640)])
            pltpu.sync_copy(out_b, den_sh.at[pl.ds(15 * 624, 640)])

        pltpu.sync_copy(tab_hbm.at[pl.ds(0, N)], h2_t)
        pltpu.sync_copy(tab_hbm.at[pl.ds(N, N)], as2_t)
        pltpu.sync_copy(tab_hbm.at[pl.ds(2 * N, N)], ad2_t)
        s0 = pl.multiple_of(sid * NC2, BLK2)
        pltpu.sync_copy(srcp_hbm.at[pl.ds(s0, BLK2)], src_v.at[pl.ds(0, BLK2)])
        pltpu.sync_copy(dstp_hbm.at[pl.ds(s0, BLK2)], dst_v.at[pl.ds(0, BLK2)])
        plsc.subcore_barrier()

        def _chunk(c, carry):
            blkrow = lax.rem(lax.div(c, BLK2), 2) * BLK2 + lax.rem(c, BLK2)
            erow = lax.rem(c, EEB)
            cw = lax.rem(c, NCHUNK2)

            @pl.when(c >= EEB)
            def _():
                pltpu.make_async_copy(
                    val_v.at[erow], num_sh.at[dst_v.at[blkrow]], sem_n).wait()
                pltpu.make_async_copy(
                    ee_v.at[erow], den_sh.at[dst_v.at[blkrow]], sem_d).wait()

            @plsc.parallel_loop(0, NG2, unroll=4)
            def _grp(g):
                o = pl.multiple_of(g * 16, 16)
                sv = src_v[blkrow, pl.ds(o, 16)]
                dv = dst_v[blkrow, pl.ds(o, 16)]
                hs = plsc.load_gather(h2_t, [sv])
                a_s = plsc.load_gather(as2_t, [sv])
                a_d = plsc.load_gather(ad2_t, [dv])
                e = a_s + a_d
                e = jnp.where(e < 0.0, e * 0.2, e)
                ee = jnp.exp(e)
                valid = (cw * CHUNK2 + o + lax.iota(jnp.int32, 16)) < EPW
                ee = jnp.where(valid, ee, 0.0)
                ee_v[erow, pl.ds(o, 16)] = ee
                val_v[erow, pl.ds(o, 16)] = ee * hs
            pltpu.async_copy(val_v.at[erow], num_sh.at[dst_v.at[blkrow]],
                             sem_n, add=True)
            pltpu.async_copy(ee_v.at[erow], den_sh.at[dst_v.at[blkrow]],
                             sem_d, add=True)

            @pl.when(jnp.logical_and(lax.rem(c + 1, BLK2) == 0, c + 1 < NC2))
            def _():
                q = c + 1
                half = pl.multiple_of(
                    lax.rem(lax.div(q, BLK2), 2) * BLK2, BLK2)
                r0 = pl.multiple_of(sid * NC2 + q, BLK2)
                pltpu.sync_copy(srcp_hbm.at[pl.ds(r0, BLK2)],
                                src_v.at[pl.ds(half, BLK2)])
                pltpu.sync_copy(dstp_hbm.at[pl.ds(r0, BLK2)],
                                dst_v.at[pl.ds(half, BLK2)])

            return carry

        lax.fori_loop(0, NC2, _chunk, 0)
        for k in range(EEB):
            pltpu.make_async_copy(val_v.at[k], num_sh.at[dst_v.at[0]],
                                  sem_n).wait()
            pltpu.make_async_copy(ee_v.at[k], den_sh.at[dst_v.at[0]],
                                  sem_d).wait()
        plsc.subcore_barrier()

        @pl.when(sid < 15)
        def _():
            pltpu.sync_copy(num_sh.at[pl.ds(d0, 624)], num_b.at[pl.ds(0, 624)])
            pltpu.sync_copy(den_sh.at[pl.ds(d0, 624)], den_b.at[pl.ds(0, 624)])

        @pl.when(sid == 15)
        def _():
            pltpu.sync_copy(num_sh.at[pl.ds(15 * 624, 640)], num_b)
            pltpu.sync_copy(den_sh.at[pl.ds(15 * 624, 640)], den_b)

        for k in range(40):
            sl = pl.ds(k * 16, 16)
            out_b[sl] = num_b[sl] / (den_b[sl] + 1e-16)

        @pl.when(sid < 15)
        def _():
            pltpu.sync_copy(out_b.at[pl.ds(0, 624)], out_hbm.at[pl.ds(d0, 624)])

        @pl.when(sid == 15)
        def _():
            pltpu.sync_copy(out_b, out_hbm.at[pl.ds(15 * 624, 640)])


def kernel(x, edge_index, W1, a1_src, a1_dst, W2, a2_src, a2_dst):
    src = edge_index[0].astype(jnp.int32)
    dst = edge_index[1].astype(jnp.int32)
    pad = jnp.broadcast_to(
        (jnp.arange(PAD_PER_W, dtype=jnp.int32) * 37) % N, (NW, PAD_PER_W))
    srcp = jnp.concatenate([src.reshape(NW, EPW), pad], axis=1)
    srcp = srcp.reshape(NW * NCHUNK, CHUNK)
    dstp = jnp.concatenate([dst.reshape(NW, EPW), pad], axis=1)
    dstp = dstp.reshape(NW * NCHUNK, CHUNK)

    h, avt = pl.pallas_call(
        _tc1_body,
        out_shape=[
            jax.ShapeDtypeStruct((N, D), jnp.float32),
            jax.ShapeDtypeStruct((2, N), jnp.float32),
        ],
    )(x, W1, a1_src, a1_dst)

    part, dens = _sc1(h, avt.reshape(2 * N), srcp, dstp)

    RB = 1000
    tab = pl.pallas_call(
        _tc2_body,
        grid=(N // RB,),
        in_specs=[
            pl.BlockSpec((2, RB, D), lambda i: (0, i, 0)),
            pl.BlockSpec((20, RB), lambda i: (0, 0)),
            pl.BlockSpec((D, 1), lambda i: (0, 0)),
            pl.BlockSpec(memory_space=pltpu.SMEM),
            pl.BlockSpec(memory_space=pltpu.SMEM),
        ],
        out_specs=pl.BlockSpec((30, RB), lambda i: (0, 0)),
        out_shape=jax.ShapeDtypeStruct((30, RB), jnp.float32),
    )(part.reshape(2, N, D), dens.reshape(20, RB), W2, a2_src, a2_dst)

    out2 = _sc2(tab.reshape(3 * N),
                srcp.reshape(NW * NCHUNK2, CHUNK2),
                dstp.reshape(NW * NCHUNK2, CHUNK2))
    return out2.reshape(1, N)
